# Initial kernel scaffold; baseline (speedup 1.0000x reference)
import jax
import jax.numpy as jnp
from jax.experimental import pallas as pl


def kernel(input_values, segment_ids):
    raise NotImplementedError("write your pallas kernel here")


# speedup vs baseline: 6.4019x; 6.4019x over previous
import functools

import jax
import jax.numpy as jnp
from jax import lax
from jax.experimental import pallas as pl
from jax.experimental.pallas import tpu as pltpu
from jax.experimental.pallas import tpu_sc as plsc

N_ROWS = 320000
N_SEG = 10000
D = 128
S_PAD = 10240
C_ROWS = 256
N_CHUNKS = N_ROWS // C_ROWS
NW = 32
MAX_CH_PER_W = -(-N_CHUNKS // NW)
STRIPE = S_PAD // 16


def _sc_body(vals, ids3, acc_out, cnt_out,
             acc_sp, cnt_sp, rows_v, ids_v, ones_v, zcnt_v):
    c = lax.axis_index("c")
    s = lax.axis_index("s")
    w = s * 2 + c

    z16 = jnp.zeros((16,), jnp.float32)
    o16 = jnp.ones((16,), jnp.float32)

    def _zrow(i, carry):
        rows_v[i // 8, pl.ds((i % 8) * 16, 16)] = z16
        return carry
    lax.fori_loop(0, 128 * 8, _zrow, 0)

    def _zcnt(i, carry):
        zcnt_v[pl.ds(i * 16, 16)] = z16
        return carry
    lax.fori_loop(0, STRIPE // 16, _zcnt, 0)

    for t in range(8):
        ones_v[pl.ds(t * 16, 16)] = o16

    base = s * STRIPE
    for q in range(STRIPE // 128):
        pltpu.sync_copy(rows_v.at[pl.ds(0, 128), :],
                        acc_sp.at[pl.ds(base + q * 128, 128), :])
    pltpu.sync_copy(zcnt_v, cnt_sp.at[pl.ds(base, STRIPE)])
    plsc.subcore_barrier()

    def _chunk(i, carry):
        k = w + i * NW

        @pl.when(k < N_CHUNKS)
        def _():
            pltpu.sync_copy(vals.at[pl.ds(k * C_ROWS, C_ROWS), :], rows_v)
            pltpu.sync_copy(ids3.at[k], ids_v)
            for j in range(C_ROWS // 128):
                pltpu.sync_copy(rows_v.at[pl.ds(j * 128, 128), :],
                                acc_sp.at[ids_v.at[j]], add=True)
                pltpu.sync_copy(ones_v, cnt_sp.at[ids_v.at[j]], add=True)
        return carry
    lax.fori_loop(0, MAX_CH_PER_W, _chunk, 0)
    plsc.subcore_barrier()

    pltpu.sync_copy(acc_sp.at[pl.ds(base, STRIPE), :],
                    acc_out.at[c, pl.ds(base, STRIPE), :])
    pltpu.sync_copy(cnt_sp.at[pl.ds(base, STRIPE)],
                    cnt_out.at[c, pl.ds(base, STRIPE)])


_sc_agg = functools.partial(
    pl.kernel,
    out_type=[
        jax.ShapeDtypeStruct((2, S_PAD, D), jnp.float32),
        jax.ShapeDtypeStruct((2, S_PAD), jnp.float32),
    ],
    mesh=plsc.VectorSubcoreMesh(core_axis_name="c", subcore_axis_name="s"),
    scratch_types=[
        pltpu.VMEM_SHARED((S_PAD, D), jnp.float32),
        pltpu.VMEM_SHARED((S_PAD,), jnp.float32),
        pltpu.VMEM((C_ROWS, D), jnp.float32),
        pltpu.VMEM((C_ROWS // 128, 128), jnp.int32),
        pltpu.VMEM((128,), jnp.float32),
        pltpu.VMEM((STRIPE,), jnp.float32),
    ],
)(_sc_body)


RB = 2048


def _merge_body(a_ref, c_ref, o_ref):
    sums = a_ref[0] + a_ref[1]
    cnts = c_ref[0] + c_ref[1]
    o_ref[...] = sums / jnp.maximum(cnts, 1.0)


_merge = pl.pallas_call(
    _merge_body,
    grid=(S_PAD // RB,),
    in_specs=[
        pl.BlockSpec((2, RB, D), lambda r: (0, r, 0)),
        pl.BlockSpec((2, RB, 1), lambda r: (0, r, 0)),
    ],
    out_specs=pl.BlockSpec((RB, D), lambda r: (r, 0)),
    out_shape=jax.ShapeDtypeStruct((S_PAD, D), jnp.float32),
)


def kernel(input_values, segment_ids):
    ids3 = segment_ids.astype(jnp.int32).reshape(N_CHUNKS, C_ROWS // 128, 128)
    acc, cnt = _sc_agg(input_values, ids3)
    y = _merge(acc, cnt.reshape(2, S_PAD, 1))
    return y[:N_SEG]


# speedup vs baseline: 8.7631x; 1.3688x over previous
import functools

import jax
import jax.numpy as jnp
from jax import lax
from jax.experimental import pallas as pl
from jax.experimental.pallas import tpu as pltpu
from jax.experimental.pallas import tpu_sc as plsc

N_ROWS = 320000
N_SEG = 10000
D = 128
S_PAD = 10240
C_ROWS = 128
N_CHUNKS = N_ROWS // C_ROWS
NW = 32
MAX_ORD = -(-N_CHUNKS // NW)
N_STEPS = -(-MAX_ORD // 2)
STRIPE = S_PAD // 16


def _sc_body(vals, ids3, acc_out, cnt_out,
             acc_sp, cnt_sp, rows0, rows1, ids0, ids1, ones_v, zcnt_v,
             semr0, semr1, semi0, semi1):
    c = lax.axis_index("c")
    s = lax.axis_index("s")
    w = s * 2 + c
    rows = (rows0, rows1)
    ids = (ids0, ids1)
    semr = (semr0, semr1)
    semi = (semi0, semi1)

    z16 = jnp.zeros((16,), jnp.float32)
    o16 = jnp.ones((16,), jnp.float32)

    def _zrow(i, carry):
        rows0[i // 8, pl.ds((i % 8) * 16, 16)] = z16
        return carry
    lax.fori_loop(0, C_ROWS * 8, _zrow, 0)

    def _zcnt(i, carry):
        zcnt_v[pl.ds(i * 16, 16)] = z16
        return carry
    lax.fori_loop(0, STRIPE // 16, _zcnt, 0)

    for t in range(8):
        ones_v[pl.ds(t * 16, 16)] = o16

    base = s * STRIPE
    for q in range(STRIPE // C_ROWS):
        pltpu.sync_copy(rows0, acc_sp.at[pl.ds(base + q * C_ROWS, C_ROWS), :])
    pltpu.sync_copy(zcnt_v, cnt_sp.at[pl.ds(base, STRIPE)])
    plsc.subcore_barrier()

    def _issue(k, b):
        @pl.when(k < N_CHUNKS)
        def _():
            pltpu.async_copy(vals.at[pl.ds(k * C_ROWS, C_ROWS), :],
                             rows[b], semr[b])
            pltpu.async_copy(ids3.at[k], ids[b], semi[b])

    _issue(w, 0)

    def _step(i, carry):
        for b in range(2):
            k = w + (2 * i + b) * NW

            @pl.when(k < N_CHUNKS)
            def _():
                _issue(k + NW, 1 - b)
                pltpu.make_async_copy(vals.at[pl.ds(k * C_ROWS, C_ROWS), :],
                                      rows[b], semr[b]).wait()
                pltpu.make_async_copy(ids3.at[k], ids[b], semi[b]).wait()
                pltpu.sync_copy(rows[b], acc_sp.at[ids[b].at[0]], add=True)
                pltpu.sync_copy(ones_v, cnt_sp.at[ids[b].at[0]], add=True)
        return carry
    lax.fori_loop(0, N_STEPS, _step, 0)
    plsc.subcore_barrier()

    pltpu.sync_copy(acc_sp.at[pl.ds(base, STRIPE), :],
                    acc_out.at[c, pl.ds(base, STRIPE), :])
    pltpu.sync_copy(cnt_sp.at[pl.ds(base, STRIPE)],
                    cnt_out.at[c, pl.ds(base, STRIPE)])


_sc_agg = functools.partial(
    pl.kernel,
    out_type=[
        jax.ShapeDtypeStruct((2, S_PAD, D), jnp.float32),
        jax.ShapeDtypeStruct((2, S_PAD), jnp.float32),
    ],
    mesh=plsc.VectorSubcoreMesh(core_axis_name="c", subcore_axis_name="s"),
    scratch_types=[
        pltpu.VMEM_SHARED((S_PAD, D), jnp.float32),
        pltpu.VMEM_SHARED((S_PAD,), jnp.float32),
        pltpu.VMEM((C_ROWS, D), jnp.float32),
        pltpu.VMEM((C_ROWS, D), jnp.float32),
        pltpu.VMEM((1, 128), jnp.int32),
        pltpu.VMEM((1, 128), jnp.int32),
        pltpu.VMEM((128,), jnp.float32),
        pltpu.VMEM((STRIPE,), jnp.float32),
        pltpu.SemaphoreType.DMA,
        pltpu.SemaphoreType.DMA,
        pltpu.SemaphoreType.DMA,
        pltpu.SemaphoreType.DMA,
    ],
)(_sc_body)


RB = 2048


def _merge_body(a_ref, c_ref, o_ref):
    sums = a_ref[0] + a_ref[1]
    cnts = c_ref[0] + c_ref[1]
    o_ref[...] = sums / jnp.maximum(cnts, 1.0)


_merge = pl.pallas_call(
    _merge_body,
    grid=(S_PAD // RB,),
    in_specs=[
        pl.BlockSpec((2, RB, D), lambda r: (0, r, 0)),
        pl.BlockSpec((2, RB, 1), lambda r: (0, r, 0)),
    ],
    out_specs=pl.BlockSpec((RB, D), lambda r: (r, 0)),
    out_shape=jax.ShapeDtypeStruct((S_PAD, D), jnp.float32),
)


def kernel(input_values, segment_ids):
    ids3 = segment_ids.astype(jnp.int32).reshape(N_CHUNKS, 1, 128)
    acc, cnt = _sc_agg(input_values, ids3)
    y = _merge(acc, cnt.reshape(2, S_PAD, 1))
    return y[:N_SEG]


# speedup vs baseline: 9.1929x; 1.0490x over previous
import functools

import jax
import jax.numpy as jnp
from jax import lax
from jax.experimental import pallas as pl
from jax.experimental.pallas import tpu as pltpu
from jax.experimental.pallas import tpu_sc as plsc

N_ROWS = 320000
N_SEG = 10000
D = 128
S_PAD = 10240
C_ROWS = 128
N_CHUNKS = N_ROWS // C_ROWS
NW = 32
MAX_ORD = -(-N_CHUNKS // NW)
N_STEPS = -(-MAX_ORD // 2)
STRIPE = S_PAD // 16


def _sc_body(vals, seg_ids, acc_out, cnt_out,
             acc_sp, cnt_sp, rows0, rows1, ids0, ids1, ones_v, zcnt_v,
             semr0, semr1, semi0, semi1, semc0, semc1):
    c = lax.axis_index("c")
    s = lax.axis_index("s")
    w = s * 2 + c
    rows = (rows0, rows1)
    ids = (ids0, ids1)
    semr = (semr0, semr1)
    semi = (semi0, semi1)
    semc = (semc0, semc1)

    z16 = jnp.zeros((16,), jnp.float32)
    o16 = jnp.ones((16,), jnp.float32)

    def _zrow(i, carry):
        rows0[i // 8, pl.ds((i % 8) * 16, 16)] = z16
        return carry
    lax.fori_loop(0, C_ROWS * 8, _zrow, 0)

    def _zcnt(i, carry):
        zcnt_v[pl.ds(i * 16, 16)] = z16
        return carry
    lax.fori_loop(0, STRIPE // 16, _zcnt, 0)

    for t in range(8):
        ones_v[pl.ds(t * 16, 16)] = o16

    base = s * STRIPE
    for q in range(STRIPE // C_ROWS):
        pltpu.sync_copy(rows0, acc_sp.at[pl.ds(base + q * C_ROWS, C_ROWS), :])
    pltpu.sync_copy(zcnt_v, cnt_sp.at[pl.ds(base, STRIPE)])
    plsc.subcore_barrier()

    def _issue(k, b):
        @pl.when(k < N_CHUNKS)
        def _():
            pltpu.async_copy(vals.at[pl.ds(k * C_ROWS, C_ROWS), :],
                             rows[b], semr[b])
            pltpu.async_copy(seg_ids.at[pl.ds(k * C_ROWS, C_ROWS)],
                             ids[b], semi[b])

    _issue(w, 0)

    def _step(i, carry):
        for b in range(2):
            ordinal = 2 * i + b
            k = w + ordinal * NW

            @pl.when(k < N_CHUNKS)
            def _():
                @pl.when(ordinal >= 1)
                def _():
                    pltpu.make_async_copy(ones_v, cnt_sp.at[ids[1 - b]],
                                          semc[1 - b]).wait()
                _issue(k + NW, 1 - b)
                pltpu.make_async_copy(vals.at[pl.ds(k * C_ROWS, C_ROWS), :],
                                      rows[b], semr[b]).wait()
                pltpu.make_async_copy(seg_ids.at[pl.ds(k * C_ROWS, C_ROWS)],
                                      ids[b], semi[b]).wait()
                pltpu.sync_copy(rows[b], acc_sp.at[ids[b]], add=True)
                pltpu.async_copy(ones_v, cnt_sp.at[ids[b]], semc[b], add=True)
        return carry
    lax.fori_loop(0, N_STEPS, _step, 0)

    n_valid = (N_CHUNKS - w + NW - 1) // NW
    last_b = (n_valid - 1) % 2

    @pl.when(last_b == 0)
    def _():
        pltpu.make_async_copy(ones_v, cnt_sp.at[ids0], semc0).wait()

    @pl.when(last_b == 1)
    def _():
        pltpu.make_async_copy(ones_v, cnt_sp.at[ids1], semc1).wait()

    plsc.subcore_barrier()

    pltpu.sync_copy(acc_sp.at[pl.ds(base, STRIPE), :],
                    acc_out.at[c, pl.ds(base, STRIPE), :])
    pltpu.sync_copy(cnt_sp.at[pl.ds(base, STRIPE)],
                    cnt_out.at[c, pl.ds(base, STRIPE)])


_sc_agg = functools.partial(
    pl.kernel,
    out_type=[
        jax.ShapeDtypeStruct((2, S_PAD, D), jnp.float32),
        jax.ShapeDtypeStruct((2, S_PAD), jnp.float32),
    ],
    mesh=plsc.VectorSubcoreMesh(core_axis_name="c", subcore_axis_name="s"),
    scratch_types=[
        pltpu.VMEM_SHARED((S_PAD, D), jnp.float32),
        pltpu.VMEM_SHARED((S_PAD,), jnp.float32),
        pltpu.VMEM((C_ROWS, D), jnp.float32),
        pltpu.VMEM((C_ROWS, D), jnp.float32),
        pltpu.VMEM((C_ROWS,), jnp.int32),
        pltpu.VMEM((C_ROWS,), jnp.int32),
        pltpu.VMEM((C_ROWS,), jnp.float32),
        pltpu.VMEM((STRIPE,), jnp.float32),
        pltpu.SemaphoreType.DMA,
        pltpu.SemaphoreType.DMA,
        pltpu.SemaphoreType.DMA,
        pltpu.SemaphoreType.DMA,
        pltpu.SemaphoreType.DMA,
        pltpu.SemaphoreType.DMA,
    ],
)(_sc_body)


RB = 2000


def _merge_body(a_ref, c_ref, o_ref):
    sums = a_ref[0] + a_ref[1]
    cnts = c_ref[0] + c_ref[1]
    o_ref[...] = sums / jnp.maximum(cnts, 1.0)


_merge = pl.pallas_call(
    _merge_body,
    grid=(N_SEG // RB,),
    in_specs=[
        pl.BlockSpec((2, RB, D), lambda r: (0, r, 0)),
        pl.BlockSpec((2, RB, 1), lambda r: (0, r, 0)),
    ],
    out_specs=pl.BlockSpec((RB, D), lambda r: (r, 0)),
    out_shape=jax.ShapeDtypeStruct((N_SEG, D), jnp.float32),
)


def kernel(input_values, segment_ids):
    acc, cnt = _sc_agg(input_values, segment_ids.astype(jnp.int32))
    return _merge(acc, cnt.reshape(2, S_PAD, 1))
